# baseline (speedup 1.0000x reference)
import functools
import jax
import jax.numpy as jnp
from jax import lax
from jax.experimental import pallas as pl
from jax.experimental.pallas import tpu as pltpu
from jax.experimental.pallas import tpu_sc as plsc

NC, NS, L = 2, 16, 16
CB = 128


def _rsqrt16(x):
    i = plsc.bitcast(x, jnp.int32)
    y = plsc.bitcast(jnp.int32(0x5F3759DF) - (i >> 1), jnp.float32)
    for _ in range(3):
        y = y * (1.5 - 0.5 * x * y * y)
    return y


def _make_deg_kernel(NP, ECP):
    RB = NP // NS
    mesh = plsc.VectorSubcoreMesh(
        core_axis_name="c", subcore_axis_name="s", num_cores=NC, num_subcores=NS
    )

    @functools.partial(
        pl.kernel,
        out_type=jax.ShapeDtypeStruct((NP,), jnp.float32),
        mesh=mesh,
        compiler_params=pltpu.CompilerParams(needs_layout_passes=False),
        scratch_types=[
            pltpu.VMEM((NP,), jnp.float32),
            pltpu.VMEM((ECP,), jnp.int32),
            pltpu.VMEM_SHARED((NS, NP), jnp.float32),
            pltpu.VMEM((RB,), jnp.float32),
            pltpu.VMEM((RB,), jnp.float32),
        ],
    )
    def deg_kernel(dstr, dis_out, deg_local, dstbuf, sh, accv, tmpv):
        sid = lax.axis_index("s")
        zeros16 = jnp.zeros((L,), jnp.float32)
        ones16 = jnp.full((L,), 1.0, jnp.float32)

        def zero_body(k, carry):
            deg_local[pl.ds(k * L, L)] = zeros16
            return carry
        lax.fori_loop(0, NP // L, zero_body, 0)

        pltpu.sync_copy(dstr.at[sid], dstbuf)

        def hist_body(k, carry):
            idx = dstbuf[pl.ds(k * L, L)]
            plsc.addupdate_scatter(deg_local, [idx], ones16)
            return carry
        lax.fori_loop(0, ECP // L, hist_body, 0)

        pltpu.sync_copy(deg_local, sh.at[sid])
        plsc.subcore_barrier()
        r0 = sid * RB
        pltpu.sync_copy(sh.at[0, pl.ds(r0, RB)], accv)

        def red_body(k, carry):
            pltpu.sync_copy(sh.at[k, pl.ds(r0, RB)], tmpv)

            def add_body(g, c2):
                accv[pl.ds(g * L, L)] = accv[pl.ds(g * L, L)] + tmpv[pl.ds(g * L, L)]
                return c2
            lax.fori_loop(0, RB // L, add_body, 0)
            return carry
        lax.fori_loop(1, NS, red_body, 0)

        def fin_body(g, carry):
            d = accv[pl.ds(g * L, L)] + 1.0
            accv[pl.ds(g * L, L)] = _rsqrt16(d)
            return carry
        lax.fori_loop(0, RB // L, fin_body, 0)
        pltpu.sync_copy(accv, dis_out.at[pl.ds(r0, RB)])

    return deg_kernel


def _matmul_scaled(x, W, dis2, NCB, NP):
    n, d = x.shape
    RT = 1000
    GR = n // RT

    def mm_body(x_ref, w_ref, s_ref, o_ref):
        o_ref[0] = (
            jnp.dot(x_ref[...], w_ref[...], preferred_element_type=jnp.float32)
            * s_ref[...]
        )

    return pl.pallas_call(
        mm_body,
        grid=(GR, NCB),
        in_specs=[
            pl.BlockSpec((RT, d), lambda i, j: (i, 0)),
            pl.BlockSpec((d, CB), lambda i, j: (0, j)),
            pl.BlockSpec((RT, 1), lambda i, j: (i, 0)),
        ],
        out_specs=pl.BlockSpec((1, RT, CB), lambda i, j: (j, i, 0)),
        out_shape=jax.ShapeDtypeStruct((NCB, NP, CB), jnp.float32),
    )(x, W, dis2)


def _make_edge_kernel(NP, ECP, NCHUNK, B3, NCB, d, n):
    RB = NP // NS
    OC = 80
    NRC = RB // OC
    NIC = RB // CB
    NBLK = NCB // NC
    mesh = plsc.VectorSubcoreMesh(
        core_axis_name="c", subcore_axis_name="s", num_cores=NC, num_subcores=NS
    )

    @functools.partial(
        pl.kernel,
        out_type=jax.ShapeDtypeStruct((n, d), jnp.float32),
        mesh=mesh,
        compiler_params=pltpu.CompilerParams(needs_layout_passes=False),
        scratch_types=[
            pltpu.VMEM((ECP,), jnp.int32),
            pltpu.VMEM((NCHUNK, B3), jnp.int32),
            pltpu.VMEM((B3, CB), jnp.float32),
            pltpu.VMEM((B3, CB), jnp.float32),
            pltpu.VMEM((OC,), jnp.float32),
            pltpu.VMEM((CB,), jnp.float32),
            pltpu.VMEM_SHARED((NP, CB), jnp.float32),
            pltpu.SemaphoreType.DMA,
            pltpu.SemaphoreType.DMA,
            pltpu.SemaphoreType.DMA,
            pltpu.SemaphoreType.DMA,
        ],
    )
    def edge_kernel(hs, srcr, dstr3, dis, b, out,
                    src_plus, dst2d, rows_a, rows_b, disv, bv, acc,
                    sem_a, sem_b, sem_sa, sem_sb):
        cid = lax.axis_index("c")
        sid = lax.axis_index("s")
        r0 = sid * RB

        def blk_body(blk, carry):
            cb = cid * NBLK + blk
            off = cb * NP

            pltpu.sync_copy(srcr.at[sid], src_plus)

            def off_body(k, c2):
                v = src_plus[pl.ds(k * L, L)]
                src_plus[pl.ds(k * L, L)] = v + off
                return c2
            lax.fori_loop(0, ECP // L, off_body, 0)
            pltpu.sync_copy(dstr3.at[sid], dst2d)

            def init_body(c, c2):
                pltpu.sync_copy(
                    hs.at[pl.ds(off + r0 + c * CB, CB)],
                    acc.at[pl.ds(r0 + c * CB, CB)],
                )
                return c2
            lax.fori_loop(0, NIC, init_body, 0)
            plsc.subcore_barrier()

            pltpu.async_copy(hs.at[src_plus.at[pl.ds(0, B3)]], rows_a, sem_a)

            def pair_body(p, c2):
                j = 2 * p

                @pl.when(p > 0)
                def _():
                    pltpu.make_async_copy(
                        rows_b, acc.at[dst2d.at[0]], sem_sb
                    ).wait()
                pltpu.async_copy(
                    hs.at[src_plus.at[pl.ds((j + 1) * B3, B3)]], rows_b, sem_b
                )
                pltpu.make_async_copy(hs.at[pl.ds(0, B3)], rows_a, sem_a).wait()
                pltpu.async_copy(rows_a, acc.at[dst2d.at[j]], sem_sa, add=True)
                pltpu.make_async_copy(hs.at[pl.ds(0, B3)], rows_b, sem_b).wait()
                pltpu.async_copy(
                    rows_b, acc.at[dst2d.at[j + 1]], sem_sb, add=True
                )
                pltpu.make_async_copy(rows_a, acc.at[dst2d.at[0]], sem_sa).wait()
                jn = lax.rem(j + 2, NCHUNK)
                pltpu.async_copy(
                    hs.at[src_plus.at[pl.ds(jn * B3, B3)]], rows_a, sem_a
                )
                return c2
            lax.fori_loop(0, NCHUNK // 2, pair_body, 0)
            pltpu.make_async_copy(rows_b, acc.at[dst2d.at[0]], sem_sb).wait()
            pltpu.make_async_copy(hs.at[pl.ds(0, B3)], rows_a, sem_a).wait()
            plsc.subcore_barrier()

            pltpu.sync_copy(b.at[pl.ds(cb * CB, CB)], bv)

            def out_body(c, c2):
                base = r0 + c * OC
                pltpu.sync_copy(acc.at[pl.ds(base, OC)], rows_a.at[pl.ds(0, OC)])
                pltpu.sync_copy(dis.at[pl.ds(base, OC)], disv)

                def row_body(gi, c3):
                    dv = disv[pl.ds(gi * L, L)]
                    for j in range(L):
                        s = dv[j]
                        i = gi * L + j
                        for g in range(CB // L):
                            v = rows_a[i, pl.ds(g * L, L)]
                            v = v * s + bv[pl.ds(g * L, L)]
                            rows_a[i, pl.ds(g * L, L)] = jnp.maximum(v, 0.0)
                    return c3
                lax.fori_loop(0, OC // L, row_body, 0)

                @pl.when(base < n)
                def _():
                    pltpu.sync_copy(
                        rows_a.at[pl.ds(0, OC)],
                        out.at[pl.ds(base, OC), pl.ds(cb * CB, CB)],
                    )
                return c2
            lax.fori_loop(0, NRC, out_body, 0)
            plsc.subcore_barrier()
            return carry
        lax.fori_loop(0, NBLK, blk_body, 0)

    return edge_kernel


def kernel(x, edge_index, W, b):
    n, d = x.shape
    e = edge_index.shape[1]
    NCB = d // CB
    RB = -(-n // NS)
    RB = -(-RB // CB) * CB
    NP = NS * RB
    B3 = 80
    NCHUNK = -(-e // (NS * B3))
    NCHUNK += NCHUNK % 2
    ECP = NCHUNK * B3
    EP = NS * ECP

    src = edge_index[0].astype(jnp.int32)
    dst = edge_index[1].astype(jnp.int32)
    pad = jnp.full((EP - e,), n, dtype=jnp.int32)
    srcp = jnp.concatenate([src, pad])
    dstp = jnp.concatenate([dst, pad])
    srcr = srcp.reshape(NS, ECP)
    dstr3 = dstp.reshape(NS, NCHUNK, B3)

    dis = _make_deg_kernel(NP, ECP)(dstp.reshape(NS, ECP))
    hs = _matmul_scaled(x, W, dis[:n].reshape(n, 1), NCB, NP)
    return _make_edge_kernel(NP, ECP, NCHUNK, B3, NCB, d, n)(
        hs.reshape(NCB * NP, CB), srcr, dstr3, dis, b
    )


# speedup vs baseline: 1.1443x; 1.1443x over previous
import functools
import jax
import jax.numpy as jnp
from jax import lax
from jax.experimental import pallas as pl
from jax.experimental.pallas import tpu as pltpu
from jax.experimental.pallas import tpu_sc as plsc

NC, NS, L = 2, 16, 16
CB = 128


def _rsqrt16(x):
    i = plsc.bitcast(x, jnp.int32)
    y = plsc.bitcast(jnp.int32(0x5F3759DF) - (i >> 1), jnp.float32)
    for _ in range(3):
        y = y * (1.5 - 0.5 * x * y * y)
    return y


def _make_deg_kernel(NP, ECP):
    RB = NP // NS
    mesh = plsc.VectorSubcoreMesh(
        core_axis_name="c", subcore_axis_name="s", num_cores=NC, num_subcores=NS
    )

    @functools.partial(
        pl.kernel,
        out_type=jax.ShapeDtypeStruct((NP,), jnp.float32),
        mesh=mesh,
        compiler_params=pltpu.CompilerParams(needs_layout_passes=False),
        scratch_types=[
            pltpu.VMEM((NP,), jnp.float32),
            pltpu.VMEM((ECP,), jnp.int32),
            pltpu.VMEM_SHARED((NS, NP), jnp.float32),
            pltpu.VMEM((RB,), jnp.float32),
            pltpu.VMEM((RB,), jnp.float32),
        ],
    )
    def deg_kernel(dstr, dis_out, deg_local, dstbuf, sh, accv, tmpv):
        sid = lax.axis_index("s")
        zeros16 = jnp.zeros((L,), jnp.float32)
        ones16 = jnp.full((L,), 1.0, jnp.float32)

        def zero_body(k, carry):
            deg_local[pl.ds(k * L, L)] = zeros16
            return carry
        lax.fori_loop(0, NP // L, zero_body, 0)

        pltpu.sync_copy(dstr.at[sid], dstbuf)

        def hist_body(k, carry):
            idx = dstbuf[pl.ds(k * L, L)]
            plsc.addupdate_scatter(deg_local, [idx], ones16)
            return carry
        lax.fori_loop(0, ECP // L, hist_body, 0)

        pltpu.sync_copy(deg_local, sh.at[sid])
        plsc.subcore_barrier()
        r0 = sid * RB
        pltpu.sync_copy(sh.at[0, pl.ds(r0, RB)], accv)

        def red_body(k, carry):
            pltpu.sync_copy(sh.at[k, pl.ds(r0, RB)], tmpv)

            def add_body(g, c2):
                accv[pl.ds(g * L, L)] = accv[pl.ds(g * L, L)] + tmpv[pl.ds(g * L, L)]
                return c2
            lax.fori_loop(0, RB // L, add_body, 0)
            return carry
        lax.fori_loop(1, NS, red_body, 0)

        def fin_body(g, carry):
            d = accv[pl.ds(g * L, L)] + 1.0
            accv[pl.ds(g * L, L)] = _rsqrt16(d)
            return carry
        lax.fori_loop(0, RB // L, fin_body, 0)
        pltpu.sync_copy(accv, dis_out.at[pl.ds(r0, RB)])

    return deg_kernel


def _matmul_scaled(x, W, dis2, NCB, NP):
    n, d = x.shape
    RT = 1000
    GR = n // RT

    def mm_body(x_ref, w_ref, s_ref, o_ref):
        o_ref[0] = (
            jnp.dot(x_ref[...], w_ref[...], preferred_element_type=jnp.float32)
            * s_ref[...]
        )

    return pl.pallas_call(
        mm_body,
        grid=(GR, NCB),
        in_specs=[
            pl.BlockSpec((RT, d), lambda i, j: (i, 0)),
            pl.BlockSpec((d, CB), lambda i, j: (0, j)),
            pl.BlockSpec((RT, 1), lambda i, j: (i, 0)),
        ],
        out_specs=pl.BlockSpec((1, RT, CB), lambda i, j: (j, i, 0)),
        out_shape=jax.ShapeDtypeStruct((NCB, NP, CB), jnp.float32),
    )(x, W, dis2)


def _make_edge_kernel(NP, ECP, NCHUNK, B3, NCB, d, n):
    RB = NP // NS
    OC = 80
    NRC = RB // OC
    NBLK = NCB // NC
    mesh = plsc.VectorSubcoreMesh(
        core_axis_name="c", subcore_axis_name="s", num_cores=NC, num_subcores=NS
    )

    @functools.partial(
        pl.kernel,
        out_type=jax.ShapeDtypeStruct((n, d), jnp.float32),
        mesh=mesh,
        compiler_params=pltpu.CompilerParams(needs_layout_passes=False),
        scratch_types=[
            pltpu.VMEM((ECP,), jnp.int32),
            pltpu.VMEM((NCHUNK, B3), jnp.int32),
            pltpu.VMEM((B3, CB), jnp.float32),
            pltpu.VMEM((B3, CB), jnp.float32),
            pltpu.VMEM((OC,), jnp.float32),
            pltpu.VMEM((CB,), jnp.float32),
            pltpu.VMEM_SHARED((NP, CB), jnp.float32),
            pltpu.SemaphoreType.DMA,
            pltpu.SemaphoreType.DMA,
        ],
    )
    def edge_kernel(hs, srcr, dstr3, dis, b, out,
                    src_plus, dst2d, rows_a, rows_b, disv, bv, acc,
                    sem_a, sem_b):
        cid = lax.axis_index("c")
        sid = lax.axis_index("s")
        r0 = sid * RB

        def blk_body(blk, carry):
            cb = cid * NBLK + blk
            off = cb * NP

            pltpu.sync_copy(srcr.at[sid], src_plus)

            def off_body(k, c2):
                v = src_plus[pl.ds(k * L, L)]
                src_plus[pl.ds(k * L, L)] = v + off
                return c2
            lax.fori_loop(0, ECP // L, off_body, 0)
            pltpu.sync_copy(dstr3.at[sid], dst2d)

            pltpu.sync_copy(
                hs.at[pl.ds(off + r0, RB)], acc.at[pl.ds(r0, RB)]
            )
            plsc.subcore_barrier()

            pltpu.async_copy(hs.at[src_plus.at[pl.ds(0, B3)]], rows_a, sem_a)

            def pair_body(p, c2):
                j = 2 * p
                pltpu.async_copy(
                    hs.at[src_plus.at[pl.ds((j + 1) * B3, B3)]], rows_b, sem_b
                )
                pltpu.make_async_copy(hs.at[pl.ds(0, B3)], rows_a, sem_a).wait()
                pltpu.sync_copy(rows_a, acc.at[dst2d.at[j]], add=True)
                jn = lax.rem(j + 2, NCHUNK)
                pltpu.async_copy(
                    hs.at[src_plus.at[pl.ds(jn * B3, B3)]], rows_a, sem_a
                )
                pltpu.make_async_copy(hs.at[pl.ds(0, B3)], rows_b, sem_b).wait()
                pltpu.sync_copy(rows_b, acc.at[dst2d.at[j + 1]], add=True)
                return c2
            lax.fori_loop(0, NCHUNK // 2, pair_body, 0)
            pltpu.make_async_copy(hs.at[pl.ds(0, B3)], rows_a, sem_a).wait()
            plsc.subcore_barrier()

            pltpu.sync_copy(b.at[pl.ds(cb * CB, CB)], bv)

            def out_body(c, c2):
                base = r0 + c * OC
                pltpu.sync_copy(acc.at[pl.ds(base, OC)], rows_a.at[pl.ds(0, OC)])
                pltpu.sync_copy(dis.at[pl.ds(base, OC)], disv)

                def row_body(gi, c3):
                    dv = disv[pl.ds(gi * L, L)]
                    for j in range(L):
                        s = dv[j]
                        i = gi * L + j
                        for g in range(CB // L):
                            v = rows_a[i, pl.ds(g * L, L)]
                            v = v * s + bv[pl.ds(g * L, L)]
                            rows_a[i, pl.ds(g * L, L)] = jnp.maximum(v, 0.0)
                    return c3
                lax.fori_loop(0, OC // L, row_body, 0)

                @pl.when(base < n)
                def _():
                    pltpu.sync_copy(
                        rows_a.at[pl.ds(0, OC)],
                        out.at[pl.ds(base, OC), pl.ds(cb * CB, CB)],
                    )
                return c2
            lax.fori_loop(0, NRC, out_body, 0)
            plsc.subcore_barrier()
            return carry
        lax.fori_loop(0, NBLK, blk_body, 0)

    return edge_kernel


def kernel(x, edge_index, W, b):
    n, d = x.shape
    e = edge_index.shape[1]
    NCB = d // CB
    RB = -(-n // NS)
    RB = -(-RB // CB) * CB
    NP = NS * RB
    B3 = 80
    NCHUNK = -(-e // (NS * B3))
    NCHUNK += NCHUNK % 2
    ECP = NCHUNK * B3
    EP = NS * ECP

    src = edge_index[0].astype(jnp.int32)
    dst = edge_index[1].astype(jnp.int32)
    pad = jnp.full((EP - e,), n, dtype=jnp.int32)
    srcp = jnp.concatenate([src, pad])
    dstp = jnp.concatenate([dst, pad])
    srcr = srcp.reshape(NS, ECP)
    dstr3 = dstp.reshape(NS, NCHUNK, B3)

    dis = _make_deg_kernel(NP, ECP)(dstp.reshape(NS, ECP))
    hs = _matmul_scaled(x, W, dis[:n].reshape(n, 1), NCB, NP)
    return _make_edge_kernel(NP, ECP, NCHUNK, B3, NCB, d, n)(
        hs.reshape(NCB * NP, CB), srcr, dstr3, dis, b
    )


# speedup vs baseline: 1.1698x; 1.0223x over previous
import functools
import jax
import jax.numpy as jnp
from jax import lax
from jax.experimental import pallas as pl
from jax.experimental.pallas import tpu as pltpu
from jax.experimental.pallas import tpu_sc as plsc

NC, NS, L = 2, 16, 16
CB = 128


def _rsqrt16(x):
    i = plsc.bitcast(x, jnp.int32)
    y = plsc.bitcast(jnp.int32(0x5F3759DF) - (i >> 1), jnp.float32)
    for _ in range(3):
        y = y * (1.5 - 0.5 * x * y * y)
    return y


def _make_deg_kernel(NP, ECP):
    RB = NP // NS
    mesh = plsc.VectorSubcoreMesh(
        core_axis_name="c", subcore_axis_name="s", num_cores=NC, num_subcores=NS
    )

    @functools.partial(
        pl.kernel,
        out_type=jax.ShapeDtypeStruct((NP,), jnp.float32),
        mesh=mesh,
        compiler_params=pltpu.CompilerParams(needs_layout_passes=False),
        scratch_types=[
            pltpu.VMEM((NP,), jnp.float32),
            pltpu.VMEM((ECP,), jnp.int32),
            pltpu.VMEM_SHARED((NS, NP), jnp.float32),
            pltpu.VMEM((RB,), jnp.float32),
            pltpu.VMEM((RB,), jnp.float32),
        ],
    )
    def deg_kernel(dstr, dis_out, deg_local, dstbuf, sh, accv, tmpv):
        sid = lax.axis_index("s")
        zeros16 = jnp.zeros((L,), jnp.float32)
        ones16 = jnp.full((L,), 1.0, jnp.float32)

        def zero_body(k, carry):
            deg_local[pl.ds(k * L, L)] = zeros16
            return carry
        lax.fori_loop(0, NP // L, zero_body, 0)

        pltpu.sync_copy(dstr.at[sid], dstbuf)

        def hist_body(k, carry):
            idx = dstbuf[pl.ds(k * L, L)]
            plsc.addupdate_scatter(deg_local, [idx], ones16)
            return carry
        lax.fori_loop(0, ECP // L, hist_body, 0)

        pltpu.sync_copy(deg_local, sh.at[sid])
        plsc.subcore_barrier()
        r0 = sid * RB
        pltpu.sync_copy(sh.at[0, pl.ds(r0, RB)], accv)

        def red_body(k, carry):
            pltpu.sync_copy(sh.at[k, pl.ds(r0, RB)], tmpv)

            def add_body(g, c2):
                accv[pl.ds(g * L, L)] = accv[pl.ds(g * L, L)] + tmpv[pl.ds(g * L, L)]
                return c2
            lax.fori_loop(0, RB // L, add_body, 0)
            return carry
        lax.fori_loop(1, NS, red_body, 0)

        def fin_body(g, carry):
            d = accv[pl.ds(g * L, L)] + 1.0
            accv[pl.ds(g * L, L)] = _rsqrt16(d)
            return carry
        lax.fori_loop(0, RB // L, fin_body, 0)
        pltpu.sync_copy(accv, dis_out.at[pl.ds(r0, RB)])

    return deg_kernel


def _matmul_scaled(x, W, dis2, NCB, NP):
    n, d = x.shape
    RT = 1000
    GR = n // RT

    def mm_body(x_ref, w_ref, s_ref, o_ref):
        o_ref[0] = (
            jnp.dot(x_ref[...], w_ref[...], preferred_element_type=jnp.float32)
            * s_ref[...]
        )

    return pl.pallas_call(
        mm_body,
        grid=(GR, NCB),
        in_specs=[
            pl.BlockSpec((RT, d), lambda i, j: (i, 0)),
            pl.BlockSpec((d, CB), lambda i, j: (0, j)),
            pl.BlockSpec((RT, 1), lambda i, j: (i, 0)),
        ],
        out_specs=pl.BlockSpec((1, RT, CB), lambda i, j: (j, i, 0)),
        out_shape=jax.ShapeDtypeStruct((NCB, NP, CB), jnp.float32),
    )(x, W, dis2)


def _make_edge_kernel(NP, ECP, NCHUNK, B3, NCB, d, n):
    RB = NP // NS
    OC = 80
    NRC = RB // OC
    NBLK = NCB // NC
    mesh = plsc.VectorSubcoreMesh(
        core_axis_name="c", subcore_axis_name="s", num_cores=NC, num_subcores=NS
    )

    @functools.partial(
        pl.kernel,
        out_type=jax.ShapeDtypeStruct((n, d), jnp.float32),
        mesh=mesh,
        compiler_params=pltpu.CompilerParams(needs_layout_passes=False),
        scratch_types=[
            pltpu.VMEM((ECP,), jnp.int32),
            pltpu.VMEM((NCHUNK, B3), jnp.int32),
            pltpu.VMEM((B3, CB), jnp.float32),
            pltpu.VMEM((B3, CB), jnp.float32),
            pltpu.VMEM((RB,), jnp.float32),
            pltpu.VMEM((CB,), jnp.float32),
            pltpu.VMEM_SHARED((NP, CB), jnp.float32),
            pltpu.SemaphoreType.DMA,
            pltpu.SemaphoreType.DMA,
        ],
    )
    def edge_kernel(hs, srcr, dstr3, dis, b, out,
                    src_plus, dst2d, rows_a, rows_b, dis_rb, bv, acc,
                    sem_a, sem_b):
        cid = lax.axis_index("c")
        sid = lax.axis_index("s")
        r0 = sid * RB
        pltpu.sync_copy(dis.at[pl.ds(r0, RB)], dis_rb)

        def blk_body(blk, carry):
            cb = cid * NBLK + blk
            off = cb * NP

            pltpu.sync_copy(srcr.at[sid], src_plus)

            def off_body(k, c2):
                v = src_plus[pl.ds(k * L, L)]
                src_plus[pl.ds(k * L, L)] = v + off
                return c2
            lax.fori_loop(0, ECP // L, off_body, 0)
            pltpu.sync_copy(dstr3.at[sid], dst2d)

            pltpu.sync_copy(
                hs.at[pl.ds(off + r0, RB)], acc.at[pl.ds(r0, RB)]
            )
            plsc.subcore_barrier()

            pltpu.async_copy(hs.at[src_plus.at[pl.ds(0, B3)]], rows_a, sem_a)

            def pair_body(p, c2):
                j = 2 * p
                pltpu.async_copy(
                    hs.at[src_plus.at[pl.ds((j + 1) * B3, B3)]], rows_b, sem_b
                )
                pltpu.make_async_copy(hs.at[pl.ds(0, B3)], rows_a, sem_a).wait()
                pltpu.sync_copy(rows_a, acc.at[dst2d.at[j]], add=True)
                jn = lax.rem(j + 2, NCHUNK)
                pltpu.async_copy(
                    hs.at[src_plus.at[pl.ds(jn * B3, B3)]], rows_a, sem_a
                )
                pltpu.make_async_copy(hs.at[pl.ds(0, B3)], rows_b, sem_b).wait()
                pltpu.sync_copy(rows_b, acc.at[dst2d.at[j + 1]], add=True)
                return c2
            lax.fori_loop(0, NCHUNK // 2, pair_body, 0)
            pltpu.make_async_copy(hs.at[pl.ds(0, B3)], rows_a, sem_a).wait()
            plsc.subcore_barrier()

            pltpu.sync_copy(b.at[pl.ds(cb * CB, CB)], bv)

            def _shade(c, buf):
                def row_body(gi, c3):
                    dv = dis_rb[pl.ds(c * OC + gi * L, L)]
                    for j in range(L):
                        s = dv[j]
                        i = gi * L + j
                        for g in range(CB // L):
                            v = buf[i, pl.ds(g * L, L)]
                            v = v * s + bv[pl.ds(g * L, L)]
                            buf[i, pl.ds(g * L, L)] = jnp.maximum(v, 0.0)
                    return c3
                lax.fori_loop(0, OC // L, row_body, 0)

            def _flush(c, buf):
                base = r0 + c * OC

                @pl.when(base < n)
                def _():
                    pltpu.sync_copy(
                        buf.at[pl.ds(0, OC)],
                        out.at[pl.ds(base, OC), pl.ds(cb * CB, CB)],
                    )

            pltpu.async_copy(
                acc.at[pl.ds(r0, OC)], rows_a.at[pl.ds(0, OC)], sem_a
            )

            def opair_body(p, c2):
                c = 2 * p
                pltpu.async_copy(
                    acc.at[pl.ds(r0 + (c + 1) * OC, OC)],
                    rows_b.at[pl.ds(0, OC)], sem_b,
                )
                pltpu.make_async_copy(
                    acc.at[pl.ds(0, OC)], rows_a.at[pl.ds(0, OC)], sem_a
                ).wait()
                _shade(c, rows_a)
                _flush(c, rows_a)
                cn = lax.rem(c + 2, NRC)
                pltpu.async_copy(
                    acc.at[pl.ds(r0 + cn * OC, OC)],
                    rows_a.at[pl.ds(0, OC)], sem_a,
                )
                pltpu.make_async_copy(
                    acc.at[pl.ds(0, OC)], rows_b.at[pl.ds(0, OC)], sem_b
                ).wait()
                _shade(c + 1, rows_b)
                _flush(c + 1, rows_b)
                return c2
            lax.fori_loop(0, NRC // 2, opair_body, 0)
            pltpu.make_async_copy(
                acc.at[pl.ds(0, OC)], rows_a.at[pl.ds(0, OC)], sem_a
            ).wait()
            plsc.subcore_barrier()
            return carry
        lax.fori_loop(0, NBLK, blk_body, 0)

    return edge_kernel


def kernel(x, edge_index, W, b):
    n, d = x.shape
    e = edge_index.shape[1]
    NCB = d // CB
    RB = -(-n // NS)
    RB = -(-RB // CB) * CB
    NP = NS * RB
    B3 = 80
    NCHUNK = -(-e // (NS * B3))
    NCHUNK += NCHUNK % 2
    ECP = NCHUNK * B3
    EP = NS * ECP

    src = edge_index[0].astype(jnp.int32)
    dst = edge_index[1].astype(jnp.int32)
    pad = jnp.full((EP - e,), n, dtype=jnp.int32)
    srcp = jnp.concatenate([src, pad])
    dstp = jnp.concatenate([dst, pad])
    srcr = srcp.reshape(NS, ECP)
    dstr3 = dstp.reshape(NS, NCHUNK, B3)

    dis = _make_deg_kernel(NP, ECP)(dstp.reshape(NS, ECP))
    hs = _matmul_scaled(x, W, dis[:n].reshape(n, 1), NCB, NP)
    return _make_edge_kernel(NP, ECP, NCHUNK, B3, NCB, d, n)(
        hs.reshape(NCB * NP, CB), srcr, dstr3, dis, b
    )


# speedup vs baseline: 1.2015x; 1.0271x over previous
import functools
import jax
import jax.numpy as jnp
from jax import lax
from jax.experimental import pallas as pl
from jax.experimental.pallas import tpu as pltpu
from jax.experimental.pallas import tpu_sc as plsc

NC, NS, L = 2, 16, 16
CB = 128


def _rsqrt16(x):
    i = plsc.bitcast(x, jnp.int32)
    y = plsc.bitcast(jnp.int32(0x5F3759DF) - (i >> 1), jnp.float32)
    for _ in range(3):
        y = y * (1.5 - 0.5 * x * y * y)
    return y


def _make_deg_kernel(NP, ECP):
    RB = NP // NS
    mesh = plsc.VectorSubcoreMesh(
        core_axis_name="c", subcore_axis_name="s", num_cores=NC, num_subcores=NS
    )

    @functools.partial(
        pl.kernel,
        out_type=jax.ShapeDtypeStruct((NP,), jnp.float32),
        mesh=mesh,
        compiler_params=pltpu.CompilerParams(needs_layout_passes=False),
        scratch_types=[
            pltpu.VMEM((NP,), jnp.float32),
            pltpu.VMEM((ECP,), jnp.int32),
            pltpu.VMEM_SHARED((NS, NP), jnp.float32),
            pltpu.VMEM((RB,), jnp.float32),
            pltpu.VMEM((RB,), jnp.float32),
        ],
    )
    def deg_kernel(dstr, dis_out, deg_local, dstbuf, sh, accv, tmpv):
        sid = lax.axis_index("s")
        zeros16 = jnp.zeros((L,), jnp.float32)
        ones16 = jnp.full((L,), 1.0, jnp.float32)

        def zero_body(k, carry):
            deg_local[pl.ds(k * L, L)] = zeros16
            return carry
        lax.fori_loop(0, NP // L, zero_body, 0)

        pltpu.sync_copy(dstr.at[sid], dstbuf)

        def hist_body(k, carry):
            for u in range(5):
                idx = dstbuf[pl.ds((k * 5 + u) * L, L)]
                plsc.addupdate_scatter(deg_local, [idx], ones16)
            return carry
        lax.fori_loop(0, ECP // L // 5, hist_body, 0)

        pltpu.sync_copy(deg_local, sh.at[sid])
        plsc.subcore_barrier()
        r0 = sid * RB
        pltpu.sync_copy(sh.at[0, pl.ds(r0, RB)], accv)

        def red_body(k, carry):
            pltpu.sync_copy(sh.at[k, pl.ds(r0, RB)], tmpv)

            def add_body(g, c2):
                accv[pl.ds(g * L, L)] = accv[pl.ds(g * L, L)] + tmpv[pl.ds(g * L, L)]
                return c2
            lax.fori_loop(0, RB // L, add_body, 0)
            return carry
        lax.fori_loop(1, NS, red_body, 0)

        def fin_body(g, carry):
            d = accv[pl.ds(g * L, L)] + 1.0
            accv[pl.ds(g * L, L)] = _rsqrt16(d)
            return carry
        lax.fori_loop(0, RB // L, fin_body, 0)
        pltpu.sync_copy(accv, dis_out.at[pl.ds(r0, RB)])

    return deg_kernel


def _matmul_scaled(x, W, dis2, NCB, NP):
    n, d = x.shape
    RT = 2000
    GR = n // RT

    def mm_body(x_ref, w_ref, s_ref, o_ref):
        o_ref[0] = (
            jnp.dot(x_ref[...], w_ref[...], preferred_element_type=jnp.float32)
            * s_ref[...]
        )

    return pl.pallas_call(
        mm_body,
        grid=(GR, NCB),
        in_specs=[
            pl.BlockSpec((RT, d), lambda i, j: (i, 0)),
            pl.BlockSpec((d, CB), lambda i, j: (0, j)),
            pl.BlockSpec((RT, 1), lambda i, j: (i, 0)),
        ],
        out_specs=pl.BlockSpec((1, RT, CB), lambda i, j: (j, i, 0)),
        out_shape=jax.ShapeDtypeStruct((NCB, NP, CB), jnp.float32),
    )(x, W, dis2)


def _make_edge_kernel(NP, ECP, NCHUNK, B3, NCB, d, n):
    RB = NP // NS
    OC = 80
    NRC = RB // OC
    NBLK = NCB // NC
    mesh = plsc.VectorSubcoreMesh(
        core_axis_name="c", subcore_axis_name="s", num_cores=NC, num_subcores=NS
    )

    @functools.partial(
        pl.kernel,
        out_type=jax.ShapeDtypeStruct((n, d), jnp.float32),
        mesh=mesh,
        compiler_params=pltpu.CompilerParams(needs_layout_passes=False),
        scratch_types=[
            pltpu.VMEM((ECP,), jnp.int32),
            pltpu.VMEM((NCHUNK, B3), jnp.int32),
            pltpu.VMEM((B3, CB), jnp.float32),
            pltpu.VMEM((B3, CB), jnp.float32),
            pltpu.VMEM((RB,), jnp.float32),
            pltpu.VMEM((CB,), jnp.float32),
            pltpu.VMEM_SHARED((NP, CB), jnp.float32),
            pltpu.SemaphoreType.DMA,
            pltpu.SemaphoreType.DMA,
        ],
    )
    def edge_kernel(hs, srcr, dstr3, dis, b, out,
                    src_plus, dst2d, rows_a, rows_b, dis_rb, bv, acc,
                    sem_a, sem_b):
        cid = lax.axis_index("c")
        sid = lax.axis_index("s")
        r0 = sid * RB
        pltpu.sync_copy(dis.at[pl.ds(r0, RB)], dis_rb)

        def blk_body(blk, carry):
            cb = cid * NBLK + blk
            off = cb * NP

            pltpu.sync_copy(srcr.at[sid], src_plus)

            def off_body(k, c2):
                v = src_plus[pl.ds(k * L, L)]
                src_plus[pl.ds(k * L, L)] = v + off
                return c2
            lax.fori_loop(0, ECP // L, off_body, 0)
            pltpu.sync_copy(dstr3.at[sid], dst2d)

            pltpu.sync_copy(
                hs.at[pl.ds(off + r0, RB)], acc.at[pl.ds(r0, RB)]
            )
            plsc.subcore_barrier()

            pltpu.async_copy(hs.at[src_plus.at[pl.ds(0, B3)]], rows_a, sem_a)

            def pair_body(p, c2):
                j = 2 * p
                pltpu.async_copy(
                    hs.at[src_plus.at[pl.ds((j + 1) * B3, B3)]], rows_b, sem_b
                )
                pltpu.make_async_copy(hs.at[pl.ds(0, B3)], rows_a, sem_a).wait()
                pltpu.sync_copy(rows_a, acc.at[dst2d.at[j]], add=True)
                jn = lax.rem(j + 2, NCHUNK)
                pltpu.async_copy(
                    hs.at[src_plus.at[pl.ds(jn * B3, B3)]], rows_a, sem_a
                )
                pltpu.make_async_copy(hs.at[pl.ds(0, B3)], rows_b, sem_b).wait()
                pltpu.sync_copy(rows_b, acc.at[dst2d.at[j + 1]], add=True)
                return c2
            lax.fori_loop(0, NCHUNK // 2, pair_body, 0)
            pltpu.make_async_copy(hs.at[pl.ds(0, B3)], rows_a, sem_a).wait()
            plsc.subcore_barrier()

            pltpu.sync_copy(b.at[pl.ds(cb * CB, CB)], bv)

            def _shade(c, buf):
                def row_body(gi, c3):
                    dv = dis_rb[pl.ds(c * OC + gi * L, L)]
                    for j in range(L):
                        s = dv[j]
                        i = gi * L + j
                        for g in range(CB // L):
                            v = buf[i, pl.ds(g * L, L)]
                            v = v * s + bv[pl.ds(g * L, L)]
                            buf[i, pl.ds(g * L, L)] = jnp.maximum(v, 0.0)
                    return c3
                lax.fori_loop(0, OC // L, row_body, 0)

            def _flush(c, buf):
                base = r0 + c * OC

                @pl.when(base < n)
                def _():
                    pltpu.sync_copy(
                        buf.at[pl.ds(0, OC)],
                        out.at[pl.ds(base, OC), pl.ds(cb * CB, CB)],
                    )

            pltpu.async_copy(
                acc.at[pl.ds(r0, OC)], rows_a.at[pl.ds(0, OC)], sem_a
            )

            def opair_body(p, c2):
                c = 2 * p
                pltpu.async_copy(
                    acc.at[pl.ds(r0 + (c + 1) * OC, OC)],
                    rows_b.at[pl.ds(0, OC)], sem_b,
                )
                pltpu.make_async_copy(
                    acc.at[pl.ds(0, OC)], rows_a.at[pl.ds(0, OC)], sem_a
                ).wait()
                _shade(c, rows_a)
                _flush(c, rows_a)
                cn = lax.rem(c + 2, NRC)
                pltpu.async_copy(
                    acc.at[pl.ds(r0 + cn * OC, OC)],
                    rows_a.at[pl.ds(0, OC)], sem_a,
                )
                pltpu.make_async_copy(
                    acc.at[pl.ds(0, OC)], rows_b.at[pl.ds(0, OC)], sem_b
                ).wait()
                _shade(c + 1, rows_b)
                _flush(c + 1, rows_b)
                return c2
            lax.fori_loop(0, NRC // 2, opair_body, 0)
            pltpu.make_async_copy(
                acc.at[pl.ds(0, OC)], rows_a.at[pl.ds(0, OC)], sem_a
            ).wait()
            plsc.subcore_barrier()
            return carry
        lax.fori_loop(0, NBLK, blk_body, 0)

    return edge_kernel


def kernel(x, edge_index, W, b):
    n, d = x.shape
    e = edge_index.shape[1]
    NCB = d // CB
    RB = -(-n // NS)
    RB = -(-RB // CB) * CB
    NP = NS * RB
    B3 = 80
    NCHUNK = -(-e // (NS * B3))
    NCHUNK += NCHUNK % 2
    ECP = NCHUNK * B3
    EP = NS * ECP

    src = edge_index[0].astype(jnp.int32)
    dst = edge_index[1].astype(jnp.int32)
    pad = jnp.full((EP - e,), n, dtype=jnp.int32)
    srcp = jnp.concatenate([src, pad])
    dstp = jnp.concatenate([dst, pad])
    srcr = srcp.reshape(NS, ECP)
    dstr3 = dstp.reshape(NS, NCHUNK, B3)

    dis = _make_deg_kernel(NP, ECP)(dstp.reshape(NS, ECP))
    hs = _matmul_scaled(x, W, dis[:n].reshape(n, 1), NCB, NP)
    return _make_edge_kernel(NP, ECP, NCHUNK, B3, NCB, d, n)(
        hs.reshape(NCB * NP, CB), srcr, dstr3, dis, b
    )


# speedup vs baseline: 1.2037x; 1.0018x over previous
import functools
import jax
import jax.numpy as jnp
from jax import lax
from jax.experimental import pallas as pl
from jax.experimental.pallas import tpu as pltpu
from jax.experimental.pallas import tpu_sc as plsc

NC, NS, L = 2, 16, 16
CB = 128


def _rsqrt16(x):
    i = plsc.bitcast(x, jnp.int32)
    y = plsc.bitcast(jnp.int32(0x5F3759DF) - (i >> 1), jnp.float32)
    for _ in range(3):
        y = y * (1.5 - 0.5 * x * y * y)
    return y


def _make_deg_kernel(NP, ECP):
    RB = NP // NS
    mesh = plsc.VectorSubcoreMesh(
        core_axis_name="c", subcore_axis_name="s", num_cores=NC, num_subcores=NS
    )

    @functools.partial(
        pl.kernel,
        out_type=jax.ShapeDtypeStruct((NP,), jnp.float32),
        mesh=mesh,
        compiler_params=pltpu.CompilerParams(needs_layout_passes=False),
        scratch_types=[
            pltpu.VMEM((NP,), jnp.float32),
            pltpu.VMEM((ECP,), jnp.int32),
            pltpu.VMEM_SHARED((NS, NP), jnp.float32),
            pltpu.VMEM((RB,), jnp.float32),
            pltpu.VMEM((RB,), jnp.float32),
        ],
    )
    def deg_kernel(dstr, dis_out, deg_local, dstbuf, sh, accv, tmpv):
        sid = lax.axis_index("s")
        zeros16 = jnp.zeros((L,), jnp.float32)
        ones16 = jnp.full((L,), 1.0, jnp.float32)

        def zero_body(k, carry):
            deg_local[pl.ds(k * L, L)] = zeros16
            return carry
        lax.fori_loop(0, NP // L, zero_body, 0)

        pltpu.sync_copy(dstr.at[sid], dstbuf)

        def hist_body(k, carry):
            for u in range(5):
                idx = dstbuf[pl.ds((k * 5 + u) * L, L)]
                plsc.addupdate_scatter(deg_local, [idx], ones16)
            return carry
        lax.fori_loop(0, ECP // L // 5, hist_body, 0)

        pltpu.sync_copy(deg_local, sh.at[sid])
        plsc.subcore_barrier()
        r0 = sid * RB
        pltpu.sync_copy(sh.at[0, pl.ds(r0, RB)], accv)

        def red_body(k, carry):
            pltpu.sync_copy(sh.at[k, pl.ds(r0, RB)], tmpv)

            def add_body(g, c2):
                accv[pl.ds(g * L, L)] = accv[pl.ds(g * L, L)] + tmpv[pl.ds(g * L, L)]
                return c2
            lax.fori_loop(0, RB // L, add_body, 0)
            return carry
        lax.fori_loop(1, NS, red_body, 0)

        def fin_body(g, carry):
            d = accv[pl.ds(g * L, L)] + 1.0
            accv[pl.ds(g * L, L)] = _rsqrt16(d)
            return carry
        lax.fori_loop(0, RB // L, fin_body, 0)
        pltpu.sync_copy(accv, dis_out.at[pl.ds(r0, RB)])

    return deg_kernel


def _matmul_scaled(x, W, dis2, NCB, NP):
    n, d = x.shape
    RT = 2000
    GR = n // RT

    def mm_body(x_ref, w_ref, s_ref, o_ref):
        o_ref[0] = (
            jnp.dot(x_ref[...], w_ref[...], preferred_element_type=jnp.float32)
            * s_ref[...]
        )

    return pl.pallas_call(
        mm_body,
        grid=(GR, NCB),
        in_specs=[
            pl.BlockSpec((RT, d), lambda i, j: (i, 0)),
            pl.BlockSpec((d, CB), lambda i, j: (0, j)),
            pl.BlockSpec((RT, 1), lambda i, j: (i, 0)),
        ],
        out_specs=pl.BlockSpec((1, RT, CB), lambda i, j: (j, i, 0)),
        out_shape=jax.ShapeDtypeStruct((NCB, NP, CB), jnp.float32),
    )(x, W, dis2)


def _make_edge_kernel(NP, ECP, NCHUNK, B3, NCB, d, n):
    RB = NP // NS
    OC = 80
    NRC = RB // OC
    NBLK = NCB // NC
    mesh = plsc.VectorSubcoreMesh(
        core_axis_name="c", subcore_axis_name="s", num_cores=NC, num_subcores=NS
    )

    @functools.partial(
        pl.kernel,
        out_type=jax.ShapeDtypeStruct((n, d), jnp.float32),
        mesh=mesh,
        compiler_params=pltpu.CompilerParams(needs_layout_passes=False),
        scratch_types=[
            pltpu.VMEM((ECP,), jnp.int32),
            pltpu.VMEM((NCHUNK, B3), jnp.int32),
            pltpu.VMEM((B3, CB), jnp.float32),
            pltpu.VMEM((B3, CB), jnp.float32),
            pltpu.VMEM((RB,), jnp.float32),
            pltpu.VMEM((CB,), jnp.float32),
            pltpu.VMEM_SHARED((NP, CB), jnp.float32),
            pltpu.SemaphoreType.DMA,
            pltpu.SemaphoreType.DMA,
        ],
    )
    def edge_kernel(hs, srcr, dstr3, dis, b, out,
                    src_plus, dst2d, rows_a, rows_b, dis_rb, bv, acc,
                    sem_a, sem_b):
        cid = lax.axis_index("c")
        sid = lax.axis_index("s")
        r0 = sid * RB
        pltpu.sync_copy(dis.at[pl.ds(r0, RB)], dis_rb)

        def blk_body(blk, carry):
            cb = cid * NBLK + blk
            off = cb * NP

            pltpu.sync_copy(srcr.at[sid], src_plus)

            def off_body(k, c2):
                v = src_plus[pl.ds(k * L, L)]
                src_plus[pl.ds(k * L, L)] = v + off
                return c2
            lax.fori_loop(0, ECP // L, off_body, 0)
            pltpu.sync_copy(dstr3.at[sid], dst2d)

            pltpu.sync_copy(
                hs.at[pl.ds(off + r0, RB)], acc.at[pl.ds(r0, RB)]
            )
            plsc.subcore_barrier()

            pltpu.async_copy(hs.at[src_plus.at[pl.ds(0, B3)]], rows_a, sem_a)

            def pair_body(p, c2):
                j = 2 * p
                pltpu.async_copy(
                    hs.at[src_plus.at[pl.ds((j + 1) * B3, B3)]], rows_b, sem_b
                )
                pltpu.make_async_copy(hs.at[pl.ds(0, B3)], rows_a, sem_a).wait()
                pltpu.sync_copy(rows_a, acc.at[dst2d.at[j]], add=True)
                jn = lax.rem(j + 2, NCHUNK)
                pltpu.async_copy(
                    hs.at[src_plus.at[pl.ds(jn * B3, B3)]], rows_a, sem_a
                )
                pltpu.make_async_copy(hs.at[pl.ds(0, B3)], rows_b, sem_b).wait()
                pltpu.sync_copy(rows_b, acc.at[dst2d.at[j + 1]], add=True)
                return c2
            lax.fori_loop(0, NCHUNK // 2, pair_body, 0)
            pltpu.make_async_copy(hs.at[pl.ds(0, B3)], rows_a, sem_a).wait()
            plsc.subcore_barrier()

            pltpu.sync_copy(b.at[pl.ds(cb * CB, CB)], bv)

            def _shade(c, buf):
                def row_body(gi, c3):
                    dv = dis_rb[pl.ds(c * OC + gi * L, L)]
                    for j in range(L):
                        s = dv[j]
                        i = gi * L + j
                        for g in range(CB // L):
                            v = buf[i, pl.ds(g * L, L)]
                            v = v * s + bv[pl.ds(g * L, L)]
                            buf[i, pl.ds(g * L, L)] = jnp.maximum(v, 0.0)
                    return c3
                lax.fori_loop(0, OC // L, row_body, 0)

            def _flush(c, buf):
                base = r0 + c * OC

                @pl.when(base < n)
                def _():
                    pltpu.sync_copy(
                        buf.at[pl.ds(0, OC)],
                        out.at[pl.ds(base, OC), pl.ds(cb * CB, CB)],
                    )

            pltpu.async_copy(
                acc.at[pl.ds(r0, OC)], rows_a.at[pl.ds(0, OC)], sem_a
            )

            def opair_body(p, c2):
                c = 2 * p
                pltpu.async_copy(
                    acc.at[pl.ds(r0 + (c + 1) * OC, OC)],
                    rows_b.at[pl.ds(0, OC)], sem_b,
                )
                pltpu.make_async_copy(
                    acc.at[pl.ds(0, OC)], rows_a.at[pl.ds(0, OC)], sem_a
                ).wait()
                _shade(c, rows_a)
                _flush(c, rows_a)
                cn = lax.rem(c + 2, NRC)
                pltpu.async_copy(
                    acc.at[pl.ds(r0 + cn * OC, OC)],
                    rows_a.at[pl.ds(0, OC)], sem_a,
                )
                pltpu.make_async_copy(
                    acc.at[pl.ds(0, OC)], rows_b.at[pl.ds(0, OC)], sem_b
                ).wait()
                _shade(c + 1, rows_b)
                _flush(c + 1, rows_b)
                return c2
            lax.fori_loop(0, NRC // 2, opair_body, 0)
            pltpu.make_async_copy(
                acc.at[pl.ds(0, OC)], rows_a.at[pl.ds(0, OC)], sem_a
            ).wait()
            plsc.subcore_barrier()
            return carry
        lax.fori_loop(0, NBLK, blk_body, 0)

    return edge_kernel


def kernel(x, edge_index, W, b):
    n, d = x.shape
    e = edge_index.shape[1]
    NCB = d // CB
    RB = -(-n // NS)
    RB = -(-RB // CB) * CB
    NP = NS * RB
    B3 = 80
    NCHUNK = -(-e // (NS * B3))
    NCHUNK += NCHUNK % 2
    ECP = NCHUNK * B3
    EP = NS * ECP

    pad = jnp.full((2, EP - e), n, dtype=jnp.int32)
    ei = jnp.concatenate([edge_index.astype(jnp.int32), pad], axis=1)
    srcr = ei[0].reshape(NS, ECP)
    dstr3 = ei[1].reshape(NS, NCHUNK, B3)

    dis = _make_deg_kernel(NP, ECP)(ei[1].reshape(NS, ECP))
    hs = _matmul_scaled(x, W, dis.reshape(NP, 1), NCB, NP)
    return _make_edge_kernel(NP, ECP, NCHUNK, B3, NCB, d, n)(
        hs.reshape(NCB * NP, CB), srcr, dstr3, dis, b
    )


# speedup vs baseline: 1.2271x; 1.0194x over previous
import functools
import jax
import jax.numpy as jnp
from jax import lax
from jax.experimental import pallas as pl
from jax.experimental.pallas import tpu as pltpu
from jax.experimental.pallas import tpu_sc as plsc

NC, NS, L = 2, 16, 16
CB = 128


def _rsqrt16(x):
    i = plsc.bitcast(x, jnp.int32)
    y = plsc.bitcast(jnp.int32(0x5F3759DF) - (i >> 1), jnp.float32)
    for _ in range(3):
        y = y * (1.5 - 0.5 * x * y * y)
    return y


def _make_deg_kernel(NP, ECP):
    RB = NP // NS
    mesh = plsc.VectorSubcoreMesh(
        core_axis_name="c", subcore_axis_name="s", num_cores=NC, num_subcores=NS
    )

    @functools.partial(
        pl.kernel,
        out_type=jax.ShapeDtypeStruct((NP,), jnp.float32),
        mesh=mesh,
        compiler_params=pltpu.CompilerParams(needs_layout_passes=False),
        scratch_types=[
            pltpu.VMEM((NP,), jnp.float32),
            pltpu.VMEM((ECP,), jnp.int32),
            pltpu.VMEM_SHARED((NS, NP), jnp.float32),
            pltpu.VMEM((RB,), jnp.float32),
            pltpu.VMEM((RB,), jnp.float32),
        ],
    )
    def deg_kernel(dstr, dis_out, deg_local, dstbuf, sh, accv, tmpv):
        sid = lax.axis_index("s")
        zeros16 = jnp.zeros((L,), jnp.float32)
        ones16 = jnp.full((L,), 1.0, jnp.float32)

        def zero_body(k, carry):
            deg_local[pl.ds(k * L, L)] = zeros16
            return carry
        lax.fori_loop(0, NP // L, zero_body, 0)

        pltpu.sync_copy(dstr.at[sid], dstbuf)

        def hist_body(k, carry):
            for u in range(5):
                idx = dstbuf[pl.ds((k * 5 + u) * L, L)]
                plsc.addupdate_scatter(deg_local, [idx], ones16)
            return carry
        lax.fori_loop(0, ECP // L // 5, hist_body, 0)

        pltpu.sync_copy(deg_local, sh.at[sid])
        plsc.subcore_barrier()
        r0 = sid * RB
        pltpu.sync_copy(sh.at[0, pl.ds(r0, RB)], accv)

        def red_body(k, carry):
            pltpu.sync_copy(sh.at[k, pl.ds(r0, RB)], tmpv)

            def add_body(g, c2):
                accv[pl.ds(g * L, L)] = accv[pl.ds(g * L, L)] + tmpv[pl.ds(g * L, L)]
                return c2
            lax.fori_loop(0, RB // L, add_body, 0)
            return carry
        lax.fori_loop(1, NS, red_body, 0)

        def fin_body(g, carry):
            d = accv[pl.ds(g * L, L)] + 1.0
            accv[pl.ds(g * L, L)] = _rsqrt16(d)
            return carry
        lax.fori_loop(0, RB // L, fin_body, 0)
        pltpu.sync_copy(accv, dis_out.at[pl.ds(r0, RB)])

    return deg_kernel


def _matmul_scaled(x, W, dis2, NCB, NP):
    n, d = x.shape
    RT = 5000
    GR = n // RT

    def mm_body(x_ref, w_ref, s_ref, o_ref):
        o_ref[0] = (
            jnp.dot(x_ref[...], w_ref[...], preferred_element_type=jnp.float32)
            * s_ref[...]
        )

    return pl.pallas_call(
        mm_body,
        grid=(GR, NCB),
        in_specs=[
            pl.BlockSpec((RT, d), lambda i, j: (i, 0)),
            pl.BlockSpec((d, CB), lambda i, j: (0, j)),
            pl.BlockSpec((RT, 1), lambda i, j: (i, 0)),
        ],
        out_specs=pl.BlockSpec((1, RT, CB), lambda i, j: (j, i, 0)),
        out_shape=jax.ShapeDtypeStruct((NCB, NP, CB), jnp.float32),
    )(x, W, dis2)


def _make_edge_kernel(NP, ECP, NCHUNK, B3, NCB, d, n):
    RB = NP // NS
    OC = 80
    NRC = RB // OC
    NBLK = NCB // NC
    mesh = plsc.VectorSubcoreMesh(
        core_axis_name="c", subcore_axis_name="s", num_cores=NC, num_subcores=NS
    )

    @functools.partial(
        pl.kernel,
        out_type=jax.ShapeDtypeStruct((n, d), jnp.float32),
        mesh=mesh,
        compiler_params=pltpu.CompilerParams(needs_layout_passes=False),
        scratch_types=[
            pltpu.VMEM((ECP,), jnp.int32),
            pltpu.VMEM((NCHUNK, B3), jnp.int32),
            pltpu.VMEM((B3, CB), jnp.float32),
            pltpu.VMEM((B3, CB), jnp.float32),
            pltpu.VMEM((RB,), jnp.float32),
            pltpu.VMEM((CB,), jnp.float32),
            pltpu.VMEM_SHARED((NP, CB), jnp.float32),
            pltpu.SemaphoreType.DMA,
            pltpu.SemaphoreType.DMA,
        ],
    )
    def edge_kernel(hs, srcr, dstr3, dis, b, out,
                    src_plus, dst2d, rows_a, rows_b, dis_rb, bv, acc,
                    sem_a, sem_b):
        cid = lax.axis_index("c")
        sid = lax.axis_index("s")
        r0 = sid * RB
        pltpu.sync_copy(dis.at[pl.ds(r0, RB)], dis_rb)

        def blk_body(blk, carry):
            cb = cid * NBLK + blk
            off = cb * NP

            pltpu.sync_copy(srcr.at[sid], src_plus)

            def off_body(k, c2):
                for u in range(5):
                    sl = pl.ds((k * 5 + u) * L, L)
                    src_plus[sl] = src_plus[sl] + off
                return c2
            lax.fori_loop(0, ECP // L // 5, off_body, 0)
            pltpu.sync_copy(dstr3.at[sid], dst2d)

            pltpu.sync_copy(
                hs.at[pl.ds(off + r0, RB)], acc.at[pl.ds(r0, RB)]
            )
            plsc.subcore_barrier()

            pltpu.async_copy(hs.at[src_plus.at[pl.ds(0, B3)]], rows_a, sem_a)

            def pair_body(p, c2):
                j = 2 * p
                pltpu.async_copy(
                    hs.at[src_plus.at[pl.ds((j + 1) * B3, B3)]], rows_b, sem_b
                )
                pltpu.make_async_copy(hs.at[pl.ds(0, B3)], rows_a, sem_a).wait()
                pltpu.sync_copy(rows_a, acc.at[dst2d.at[j]], add=True)
                jn = lax.rem(j + 2, NCHUNK)
                pltpu.async_copy(
                    hs.at[src_plus.at[pl.ds(jn * B3, B3)]], rows_a, sem_a
                )
                pltpu.make_async_copy(hs.at[pl.ds(0, B3)], rows_b, sem_b).wait()
                pltpu.sync_copy(rows_b, acc.at[dst2d.at[j + 1]], add=True)
                return c2
            lax.fori_loop(0, NCHUNK // 2, pair_body, 0)
            pltpu.make_async_copy(hs.at[pl.ds(0, B3)], rows_a, sem_a).wait()
            plsc.subcore_barrier()

            pltpu.sync_copy(b.at[pl.ds(cb * CB, CB)], bv)

            def _shade(c, buf):
                def row_body(gi, c3):
                    dv = dis_rb[pl.ds(c * OC + gi * L, L)]
                    for j in range(L):
                        s = dv[j]
                        i = gi * L + j
                        for g in range(CB // L):
                            v = buf[i, pl.ds(g * L, L)]
                            v = v * s + bv[pl.ds(g * L, L)]
                            buf[i, pl.ds(g * L, L)] = jnp.maximum(v, 0.0)
                    return c3
                lax.fori_loop(0, OC // L, row_body, 0)

            def _flush(c, buf):
                base = r0 + c * OC

                @pl.when(base < n)
                def _():
                    pltpu.sync_copy(
                        buf.at[pl.ds(0, OC)],
                        out.at[pl.ds(base, OC), pl.ds(cb * CB, CB)],
                    )

            pltpu.async_copy(
                acc.at[pl.ds(r0, OC)], rows_a.at[pl.ds(0, OC)], sem_a
            )

            def opair_body(p, c2):
                c = 2 * p
                pltpu.async_copy(
                    acc.at[pl.ds(r0 + (c + 1) * OC, OC)],
                    rows_b.at[pl.ds(0, OC)], sem_b,
                )
                pltpu.make_async_copy(
                    acc.at[pl.ds(0, OC)], rows_a.at[pl.ds(0, OC)], sem_a
                ).wait()
                _shade(c, rows_a)
                _flush(c, rows_a)
                cn = lax.rem(c + 2, NRC)
                pltpu.async_copy(
                    acc.at[pl.ds(r0 + cn * OC, OC)],
                    rows_a.at[pl.ds(0, OC)], sem_a,
                )
                pltpu.make_async_copy(
                    acc.at[pl.ds(0, OC)], rows_b.at[pl.ds(0, OC)], sem_b
                ).wait()
                _shade(c + 1, rows_b)
                _flush(c + 1, rows_b)
                return c2
            lax.fori_loop(0, NRC // 2, opair_body, 0)
            pltpu.make_async_copy(
                acc.at[pl.ds(0, OC)], rows_a.at[pl.ds(0, OC)], sem_a
            ).wait()
            plsc.subcore_barrier()
            return carry
        lax.fori_loop(0, NBLK, blk_body, 0)

    return edge_kernel


def kernel(x, edge_index, W, b):
    n, d = x.shape
    e = edge_index.shape[1]
    NCB = d // CB
    RB = -(-n // NS)
    RB = -(-RB // CB) * CB
    NP = NS * RB
    B3 = 80
    NCHUNK = -(-e // (NS * B3))
    NCHUNK += NCHUNK % 2
    ECP = NCHUNK * B3
    EP = NS * ECP

    pad = jnp.full((2, EP - e), n, dtype=jnp.int32)
    ei = jnp.concatenate([edge_index.astype(jnp.int32), pad], axis=1)
    srcr = ei[0].reshape(NS, ECP)
    dstr3 = ei[1].reshape(NS, NCHUNK, B3)

    dis = _make_deg_kernel(NP, ECP)(ei[1].reshape(NS, ECP))
    hs = _matmul_scaled(x, W, dis.reshape(NP, 1), NCB, NP)
    return _make_edge_kernel(NP, ECP, NCHUNK, B3, NCB, d, n)(
        hs.reshape(NCB * NP, CB), srcr, dstr3, dis, b
    )
